# Initial kernel scaffold; baseline (speedup 1.0000x reference)
import jax
import jax.numpy as jnp
from jax.experimental import pallas as pl


def kernel(wt_pos14, wt_aa, wt_seq, wt_phys, wt_crg, wt_chain, wt_mask14, mut_pos14, mut_aa, mut_seq, mut_phys, mut_crg, mut_chain, mut_mask14, relpos_emb, aa_emb, W_coord, b_coord, Wq, Wk, Wv, Wo, w_pair, W1, b1, W2, b2, W3, b3, W4, b4, Wp):
    raise NotImplementedError("write your pallas kernel here")


# speedup vs baseline: 7.5412x; 7.5412x over previous
import functools
import math

import jax
import jax.numpy as jnp
from jax.experimental import pallas as pl

N, L, F, P = 4, 256, 128, 64
MAX_RELPOS = 32
NLAYERS = 2

_F32 = jnp.float32


def _ddg_kernel(wt_pos_ref, mut_pos_ref, wt_cat_ref, mut_cat_ref,
                wt_aa_ref, wt_seq_ref, wt_chain_ref,
                mut_aa_ref, mut_seq_ref, mut_chain_ref,
                aaE_ref, Wc_ref, bc_ref,
                Wq_ref, Wk_ref, Wv_ref, Wo_ref,
                M_ref, cpair_ref,
                W1_ref, b1_ref, W2_ref, b2_ref, W3_ref, b3_ref,
                W4_ref, b4_ref, Wp_ref,
                out_ref):
    inv_sqrt_f = 1.0 / math.sqrt(float(F))

    def dot(a, b, dims):
        return jax.lax.dot_general(a, b, (dims, ((), ())),
                                   preferred_element_type=_F32)

    def pdot(a, b, dims):
        return jax.lax.dot_general(a, b, (dims, ((), ())),
                                   precision=jax.lax.Precision.HIGHEST,
                                   preferred_element_type=_F32)

    def encode(pos_ref, cat_ref, aa_ref, seq_ref, chain_ref):
        coords = pos_ref[0]
        aa_row = aa_ref[0]
        seq_row = seq_ref[0]
        chain_row = chain_ref[0]

        iota32 = jax.lax.broadcasted_iota(jnp.int32, (32, L), 0)
        A = (aa_row == iota32).astype(_F32)
        emb0 = pdot(A, aaE_ref[...], (((0,), (0,))))
        res = emb0 + jax.nn.relu(
            dot(coords, Wc_ref[...], (((1,), (0,)))) + bc_ref[...])

        ca_t = cat_ref[0]
        dx = coords[:, 3:4] - ca_t[0:1, :]
        dy = coords[:, 4:5] - ca_t[1:2, :]
        dz = coords[:, 5:6] - ca_t[2:3, :]
        d = jnp.sqrt(dx * dx + dy * dy + dz * dz + 1e-8)

        iota8 = jax.lax.broadcasted_iota(jnp.int32, (8, L), 0)
        C = (chain_row == iota8).astype(_F32)
        same = pdot(C, C, (((0,), (0,))))

        iotaL = jax.lax.broadcasted_iota(jnp.int32, (L, L), 0)
        S = (seq_row == iotaL).astype(_F32)
        same_b = same > 0.5
        pair_bias = []
        for l in range(NLAYERS):
            T1 = pdot(S, M_ref[l], (((0,), (0,))))
            bias = pdot(T1, S, (((1,), (0,))))
            c = cpair_ref[l:l + 1, 0:1]
            pair_bias.append(jnp.where(same_b, bias, c))

        for l in range(NLAYERS):
            q = dot(res, Wq_ref[l], (((1,), (0,))))
            k = dot(res, Wk_ref[l], (((1,), (0,))))
            v = dot(res, Wv_ref[l], (((1,), (0,))))
            logits = dot(q, k, (((1,), (1,)))) * inv_sqrt_f
            logits = logits + pair_bias[l] - 0.1 * d
            m = jnp.max(logits, axis=-1, keepdims=True)
            e = jnp.exp(logits - m)
            s = jnp.sum(e, axis=-1, keepdims=True)
            av = dot(e, v, (((1,), (0,)))) / s
            upd = dot(av, Wo_ref[l], (((1,), (0,))))
            res = res + jax.nn.relu(upd)
        return res

    feat_wt = encode(wt_pos_ref, wt_cat_ref, wt_aa_ref, wt_seq_ref,
                     wt_chain_ref)
    feat_mut = encode(mut_pos_ref, mut_cat_ref, mut_aa_ref, mut_seq_ref,
                      mut_chain_ref)

    W1a = W1_ref[0:F]
    W1b = W1_ref[F:2 * F]

    def mlp(a, b):
        h = jax.nn.relu(dot(a, W1a, (((1,), (0,))))
                        + dot(b, W1b, (((1,), (0,)))) + b1_ref[...])
        h = jax.nn.relu(dot(h, W2_ref[...], (((1,), (0,)))) + b2_ref[...])
        h = jax.nn.relu(dot(h, W3_ref[...], (((1,), (0,)))) + b3_ref[...])
        return dot(h, W4_ref[...], (((1,), (0,)))) + b4_ref[...]

    diff = mlp(feat_wt, feat_mut) - mlp(feat_mut, feat_wt)
    per_res = dot(diff, Wp_ref[...], (((1,), (0,))))
    out_ref[0] = jnp.sum(per_res, axis=0, keepdims=True)


@jax.jit
def kernel(wt_pos14, wt_aa, wt_seq, wt_phys, wt_crg, wt_chain, wt_mask14,
           mut_pos14, mut_aa, mut_seq, mut_phys, mut_crg, mut_chain, mut_mask14,
           relpos_emb, aa_emb, W_coord, b_coord, Wq, Wk, Wv, Wo, w_pair,
           W1, b1, W2, b2, W3, b3, W4, b4, Wp):
    del wt_phys, wt_crg, wt_mask14, mut_phys, mut_crg, mut_mask14

    table = w_pair @ relpos_emb.T
    idx = jnp.clip(jnp.arange(L)[None, :] - jnp.arange(L)[:, None],
                   -MAX_RELPOS, MAX_RELPOS) + MAX_RELPOS
    M = table[:, idx]
    cpair = jnp.broadcast_to(table[:, 2 * MAX_RELPOS + 1][:, None],
                             (NLAYERS, 128))
    aaE = jnp.zeros((32, F), _F32).at[:aa_emb.shape[0]].set(aa_emb)

    wt_pos = wt_pos14.reshape(N, L, 42)
    mut_pos = mut_pos14.reshape(N, L, 42)
    wt_cat = wt_pos14[:, :, 1].transpose(0, 2, 1)
    mut_cat = mut_pos14[:, :, 1].transpose(0, 2, 1)
    ints = [x.astype(jnp.int32).reshape(N, 1, L)
            for x in (wt_aa, wt_seq, wt_chain, mut_aa, mut_seq, mut_chain)]

    def full(shape):
        nd = len(shape)
        return pl.BlockSpec(shape, lambda n, _nd=nd: (0,) * _nd)

    per_cplx = pl.BlockSpec((1, L, 42), lambda n: (n, 0, 0))
    per_cat = pl.BlockSpec((1, 3, L), lambda n: (n, 0, 0))
    per_int = pl.BlockSpec((1, 1, L), lambda n: (n, 0, 0))

    out = pl.pallas_call(
        _ddg_kernel,
        grid=(N,),
        in_specs=[per_cplx, per_cplx, per_cat, per_cat,
                  per_int, per_int, per_int, per_int, per_int, per_int,
                  full((32, F)), full((42, F)), full((1, F)),
                  full((NLAYERS, F, F)), full((NLAYERS, F, F)),
                  full((NLAYERS, F, F)), full((NLAYERS, F, F)),
                  full((NLAYERS, L, L)), full((NLAYERS, 128)),
                  full((2 * F, F)), full((1, F)),
                  full((F, F)), full((1, F)),
                  full((F, F)), full((1, F)),
                  full((F, F)), full((1, F)),
                  full((F, 1))],
        out_specs=pl.BlockSpec((1, 1, 1), lambda n: (n, 0, 0)),
        out_shape=jax.ShapeDtypeStruct((N, 1, 1), _F32),
    )(wt_pos, mut_pos, wt_cat, mut_cat, *ints,
      aaE, W_coord, b_coord.reshape(1, F),
      Wq, Wk, Wv, Wo,
      M, cpair,
      W1, b1.reshape(1, F), W2, b2.reshape(1, F), W3, b3.reshape(1, F),
      W4, b4.reshape(1, F), Wp.reshape(F, 1))
    return out.reshape(N)


# speedup vs baseline: 7.5459x; 1.0006x over previous
import functools
import math

import jax
import jax.numpy as jnp
from jax.experimental import pallas as pl
from jax.experimental.pallas import tpu as pltpu

N, L, F, P = 4, 256, 128, 64
MAX_RELPOS = 32
NLAYERS = 2

_F32 = jnp.float32


def _ddg_kernel(wt_pos_ref, mut_pos_ref, wt_cat_ref, mut_cat_ref,
                wt_aa_ref, wt_seq_ref, wt_chain_ref,
                mut_aa_ref, mut_seq_ref, mut_chain_ref,
                aaE_ref, Wc_ref, bc_ref,
                Wq_ref, Wk_ref, Wv_ref, Wo_ref,
                M_ref, cpair_ref,
                W1_ref, b1_ref, W2_ref, b2_ref, W3_ref, b3_ref,
                W4_ref, b4_ref, Wp_ref,
                out_ref):
    inv_sqrt_f = 1.0 / math.sqrt(float(F))

    def dot(a, b, dims):
        return jax.lax.dot_general(a, b, (dims, ((), ())),
                                   preferred_element_type=_F32)

    def pdot(a, b, dims):
        return jax.lax.dot_general(a, b, (dims, ((), ())),
                                   precision=jax.lax.Precision.HIGHEST,
                                   preferred_element_type=_F32)

    def encode(pos_ref, cat_ref, aa_ref, seq_ref, chain_ref):
        coords = pos_ref[0]
        aa_row = aa_ref[0]
        seq_row = seq_ref[0]
        chain_row = chain_ref[0]

        iota32 = jax.lax.broadcasted_iota(jnp.int32, (32, L), 0)
        A = (aa_row == iota32).astype(_F32)
        emb0 = pdot(A, aaE_ref[...], (((0,), (0,))))
        res = emb0 + jax.nn.relu(
            dot(coords, Wc_ref[...], (((1,), (0,)))) + bc_ref[...])

        ca_t = cat_ref[0]
        dx = coords[:, 3:4] - ca_t[0:1, :]
        dy = coords[:, 4:5] - ca_t[1:2, :]
        dz = coords[:, 5:6] - ca_t[2:3, :]
        d = jnp.sqrt(dx * dx + dy * dy + dz * dz + 1e-8)

        iota8 = jax.lax.broadcasted_iota(jnp.int32, (8, L), 0)
        C = (chain_row == iota8).astype(_F32)
        same = pdot(C, C, (((0,), (0,))))

        iotaL = jax.lax.broadcasted_iota(jnp.int32, (L, L), 0)
        S = (seq_row == iotaL).astype(_F32)
        same_b = same > 0.5
        pair_bias = []
        for l in range(NLAYERS):
            T1 = pdot(S, M_ref[l], (((0,), (0,))))
            bias = pdot(T1, S, (((1,), (0,))))
            c = cpair_ref[l:l + 1, 0:1]
            pair_bias.append(jnp.where(same_b, bias, c))

        for l in range(NLAYERS):
            q = dot(res, Wq_ref[l], (((1,), (0,))))
            k = dot(res, Wk_ref[l], (((1,), (0,))))
            v = dot(res, Wv_ref[l], (((1,), (0,))))
            logits = dot(q, k, (((1,), (1,)))) * inv_sqrt_f
            logits = logits + pair_bias[l] - 0.1 * d
            m = jnp.max(logits, axis=-1, keepdims=True)
            e = jnp.exp(logits - m)
            s = jnp.sum(e, axis=-1, keepdims=True)
            av = dot(e, v, (((1,), (0,)))) / s
            upd = dot(av, Wo_ref[l], (((1,), (0,))))
            res = res + jax.nn.relu(upd)
        return res

    feat_wt = encode(wt_pos_ref, wt_cat_ref, wt_aa_ref, wt_seq_ref,
                     wt_chain_ref)
    feat_mut = encode(mut_pos_ref, mut_cat_ref, mut_aa_ref, mut_seq_ref,
                      mut_chain_ref)

    W1a = W1_ref[0:F]
    W1b = W1_ref[F:2 * F]

    def mlp(a, b):
        h = jax.nn.relu(dot(a, W1a, (((1,), (0,))))
                        + dot(b, W1b, (((1,), (0,)))) + b1_ref[...])
        h = jax.nn.relu(dot(h, W2_ref[...], (((1,), (0,)))) + b2_ref[...])
        h = jax.nn.relu(dot(h, W3_ref[...], (((1,), (0,)))) + b3_ref[...])
        return dot(h, W4_ref[...], (((1,), (0,)))) + b4_ref[...]

    diff = mlp(feat_wt, feat_mut) - mlp(feat_mut, feat_wt)
    per_res = dot(diff, Wp_ref[...], (((1,), (0,))))
    out_ref[0] = jnp.sum(per_res, axis=0, keepdims=True)


@jax.jit
def kernel(wt_pos14, wt_aa, wt_seq, wt_phys, wt_crg, wt_chain, wt_mask14,
           mut_pos14, mut_aa, mut_seq, mut_phys, mut_crg, mut_chain, mut_mask14,
           relpos_emb, aa_emb, W_coord, b_coord, Wq, Wk, Wv, Wo, w_pair,
           W1, b1, W2, b2, W3, b3, W4, b4, Wp):
    del wt_phys, wt_crg, wt_mask14, mut_phys, mut_crg, mut_mask14

    table = w_pair @ relpos_emb.T
    idx = jnp.clip(jnp.arange(L)[None, :] - jnp.arange(L)[:, None],
                   -MAX_RELPOS, MAX_RELPOS) + MAX_RELPOS
    M = table[:, idx]
    cpair = jnp.broadcast_to(table[:, 2 * MAX_RELPOS + 1][:, None],
                             (NLAYERS, 128))
    aaE = jnp.zeros((32, F), _F32).at[:aa_emb.shape[0]].set(aa_emb)

    wt_pos = wt_pos14.reshape(N, L, 42)
    mut_pos = mut_pos14.reshape(N, L, 42)
    wt_cat = wt_pos14[:, :, 1].transpose(0, 2, 1)
    mut_cat = mut_pos14[:, :, 1].transpose(0, 2, 1)
    ints = [x.astype(jnp.int32).reshape(N, 1, L)
            for x in (wt_aa, wt_seq, wt_chain, mut_aa, mut_seq, mut_chain)]

    def full(shape):
        nd = len(shape)
        return pl.BlockSpec(shape, lambda n, _nd=nd: (0,) * _nd)

    per_cplx = pl.BlockSpec((1, L, 42), lambda n: (n, 0, 0))
    per_cat = pl.BlockSpec((1, 3, L), lambda n: (n, 0, 0))
    per_int = pl.BlockSpec((1, 1, L), lambda n: (n, 0, 0))

    out = pl.pallas_call(
        _ddg_kernel,
        grid=(N,),
        in_specs=[per_cplx, per_cplx, per_cat, per_cat,
                  per_int, per_int, per_int, per_int, per_int, per_int,
                  full((32, F)), full((42, F)), full((1, F)),
                  full((NLAYERS, F, F)), full((NLAYERS, F, F)),
                  full((NLAYERS, F, F)), full((NLAYERS, F, F)),
                  full((NLAYERS, L, L)), full((NLAYERS, 128)),
                  full((2 * F, F)), full((1, F)),
                  full((F, F)), full((1, F)),
                  full((F, F)), full((1, F)),
                  full((F, F)), full((1, F)),
                  full((F, 1))],
        out_specs=pl.BlockSpec((1, 1, 1), lambda n: (n, 0, 0)),
        out_shape=jax.ShapeDtypeStruct((N, 1, 1), _F32),
        compiler_params=pltpu.CompilerParams(
            dimension_semantics=("parallel",)),
    )(wt_pos, mut_pos, wt_cat, mut_cat, *ints,
      aaE, W_coord, b_coord.reshape(1, F),
      Wq, Wk, Wv, Wo,
      M, cpair,
      W1, b1.reshape(1, F), W2, b2.reshape(1, F), W3, b3.reshape(1, F),
      W4, b4.reshape(1, F), Wp.reshape(F, 1))
    return out.reshape(N)
